# baseline (speedup 1.0000x reference)
import jax
import jax.numpy as jnp
from jax import lax
from jax.experimental import pallas as pl

BT = 256
NROWS = 4096
CBN = 8192
CBD = 32
BETA = 0.25
GRID = NROWS // BT
LOSS_SCALE = (1.0 + BETA) / (3.0 * NROWS * CBD)


def _mm(a, b):
    return lax.dot_general(a, b, (((1,), (0,)), ((), ())),
                           preferred_element_type=jnp.float32)


def _fused_kernel(x_ref, ew0, eb0, ew1, eb1, ew2, eb2, ew3, eb3,
                  dw0, db0, dw1, db1, dw2, db2, dw3, db3,
                  cb0, cb1, cb2, cbt0, cbt1, cbt2,
                  out_ref, loss_ref, idx_ref):
    i = pl.program_id(0)

    h = x_ref[...]
    h = jnp.maximum(_mm(h, ew0[...]) + eb0[...], 0.0)
    h = jnp.maximum(_mm(h, ew1[...]) + eb1[...], 0.0)
    h = jnp.maximum(_mm(h, ew2[...]) + eb2[...], 0.0)
    z = _mm(h, ew3[...]) + eb3[...]

    res = z
    xqsum = jnp.zeros_like(z)
    ssq = jnp.zeros((1, 1), jnp.float32)
    cols = lax.broadcasted_iota(jnp.int32, (BT, CBN), 1)
    idx_parts = []
    for cb, cbt in ((cb0, cbt0), (cb1, cbt1), (cb2, cbt2)):
        rsq = jnp.sum(res * res, axis=1, keepdims=True)
        cbsq = jnp.sum(cbt[...] * cbt[...], axis=0, keepdims=True)
        zz = _mm(res, cbt[...])
        d = (rsq + cbsq) - 2.0 * zz
        m = jnp.min(d, axis=1, keepdims=True)
        idxv = jnp.min(jnp.where(d == m, cols, CBN), axis=1, keepdims=True)
        onehot = (cols == idxv).astype(jnp.float32)
        xq = _mm(onehot, cb[...])
        err = xq - res
        ssq = ssq + jnp.sum(err * err, keepdims=True)
        x_res = res + (xq - res)
        res = res - x_res
        xqsum = xqsum + x_res
        idx_parts.append(idxv)

    idx_ref[...] = jnp.concatenate(idx_parts, axis=1)

    h = jnp.maximum(_mm(xqsum, dw0[...]) + db0[...], 0.0)
    h = jnp.maximum(_mm(h, dw1[...]) + db1[...], 0.0)
    h = jnp.maximum(_mm(h, dw2[...]) + db2[...], 0.0)
    out_ref[...] = _mm(h, dw3[...]) + db3[...]

    prev = jnp.where(i == 0, jnp.zeros((1, 1), jnp.float32), loss_ref[...])
    tot = prev + ssq
    loss_ref[...] = jnp.where(i == GRID - 1, tot * LOSS_SCALE, tot)


def _full(shape):
    nd = len(shape)
    return pl.BlockSpec(shape, lambda i, _n=nd: (0,) * _n)


def kernel(x, enc_W0, enc_b0, enc_W1, enc_b1, enc_W2, enc_b2, enc_W3,
           enc_b3, dec_W0, dec_b0, dec_W1, dec_b1, dec_W2, dec_b2, dec_W3,
           dec_b3, cb0, cb1, cb2):
    ews = [enc_W0.T, enc_W1.T, enc_W2.T, enc_W3.T]
    ebs = [enc_b0.reshape(1, -1), enc_b1.reshape(1, -1),
           enc_b2.reshape(1, -1), enc_b3.reshape(1, -1)]
    dws = [dec_W0.T, dec_W1.T, dec_W2.T, dec_W3.T]
    dbs = [dec_b0.reshape(1, -1), dec_b1.reshape(1, -1),
           dec_b2.reshape(1, -1), dec_b3.reshape(1, -1)]
    cbts = [cb0.T, cb1.T, cb2.T]

    operands = []
    in_specs = [pl.BlockSpec((BT, 768), lambda i: (i, 0))]
    operands.append(x)
    for w, b in zip(ews, ebs):
        operands += [w, b]
        in_specs += [_full(w.shape), _full(b.shape)]
    for w, b in zip(dws, dbs):
        operands += [w, b]
        in_specs += [_full(w.shape), _full(b.shape)]
    for cb in (cb0, cb1, cb2):
        operands.append(cb)
        in_specs.append(_full(cb.shape))
    for cbt in cbts:
        operands.append(cbt)
        in_specs.append(_full(cbt.shape))

    out, loss, idx = pl.pallas_call(
        _fused_kernel,
        grid=(GRID,),
        in_specs=in_specs,
        out_specs=[
            pl.BlockSpec((BT, 768), lambda i: (i, 0)),
            pl.BlockSpec((1, 1), lambda i: (0, 0)),
            pl.BlockSpec((BT, 3), lambda i: (i, 0)),
        ],
        out_shape=[
            jax.ShapeDtypeStruct((NROWS, 768), jnp.float32),
            jax.ShapeDtypeStruct((1, 1), jnp.float32),
            jax.ShapeDtypeStruct((NROWS, 3), jnp.int32),
        ],
    )(*operands)
    return (out, loss.reshape(()), idx)
